# baseline (speedup 1.0000x reference)
import functools

import jax
import jax.numpy as jnp
from jax import lax
from jax.experimental import pallas as pl
from jax.experimental.pallas import tpu as pltpu
from jax.experimental.pallas import tpu_sc as plsc

NUM_EMB = 100000
EMB_DIM = 32
BATCH = 16384

_G = 25088
_NROW = _G
_LSTEP = 256
_TGRID = _G // _LSTEP

_INFO = plsc.get_sparse_core_info()
_NC = _INFO.num_cores
_NS = _INFO.num_subcores
_NW = _NC * _NS
_B_PER_W = BATCH // _NW
_GCHUNK = 128
_NGATHER = _B_PER_W // _GCHUNK


def _retile_body(t0, t1, t2, t3, out_ref):
    ident = (
        lax.broadcasted_iota(jnp.int32, (EMB_DIM, EMB_DIM), 0)
        == lax.broadcasted_iota(jnp.int32, (EMB_DIM, EMB_DIM), 1)
    ).astype(jnp.float32)
    parts = [
        lax.dot_general(
            r[...], ident, (((0,), (0,)), ((), ())),
            precision=lax.Precision.HIGHEST,
        )
        for r in (t0, t1, t2, t3)
    ]
    out_ref[...] = jnp.concatenate(parts, axis=1)


_retile = pl.pallas_call(
    _retile_body,
    grid=(_TGRID,),
    in_specs=[
        pl.BlockSpec((EMB_DIM, _LSTEP), functools.partial(lambda j, a: (0, j * _TGRID + a), j))
        for j in range(4)
    ],
    out_specs=pl.BlockSpec((_LSTEP, 128), lambda a: (a, 0)),
    out_shape=jax.ShapeDtypeStruct((_NROW, 128), jnp.float32),
)


@functools.partial(
    pl.kernel,
    mesh=plsc.VectorSubcoreMesh(core_axis_name="c", subcore_axis_name="s"),
    out_type=jax.ShapeDtypeStruct((BATCH, 128), jnp.float32),
    scratch_types=[
        pltpu.VMEM((_B_PER_W,), jnp.int32),
        pltpu.VMEM((_NGATHER, _GCHUNK), jnp.int32),
        pltpu.VMEM((_B_PER_W, 128), jnp.float32),
        pltpu.SemaphoreType.DMA,
    ],
)
def _embed_lookup(idx_hbm, tab_hbm, out_hbm, idx_v, q_v, rows_v, sem):
    wid = lax.axis_index("s") * _NC + lax.axis_index("c")
    base = wid * _B_PER_W
    pltpu.sync_copy(idx_hbm.at[pl.ds(base, _B_PER_W)], idx_v)

    for k in range(_B_PER_W // 16):
        v = idx_v[pl.ds(16 * k, 16)]
        q = jnp.where(
            v >= 3 * _G,
            v - 3 * _G,
            jnp.where(v >= 2 * _G, v - 2 * _G, jnp.where(v >= _G, v - _G, v)),
        )
        q_v[k // 8, pl.ds(16 * (k % 8), 16)] = q

    copies = [
        pltpu.async_copy(
            tab_hbm.at[q_v.at[j]], rows_v.at[pl.ds(_GCHUNK * j, _GCHUNK)], sem
        )
        for j in range(_NGATHER)
    ]
    for cp in copies:
        cp.wait()

    pltpu.sync_copy(rows_v, out_hbm.at[pl.ds(base, _B_PER_W)])


def kernel(char_indices, table):
    idx = char_indices.astype(jnp.int32)
    tt = table.T
    tab = _retile(tt, tt, tt, tt)
    wide = _embed_lookup(idx, tab)
    grp = (
        (idx >= _G).astype(jnp.int32)
        + (idx >= 2 * _G).astype(jnp.int32)
        + (idx >= 3 * _G).astype(jnp.int32)
    )[:, None]
    out = wide[:, 0:EMB_DIM]
    for j in range(1, 4):
        out = jnp.where(grp == j, wide[:, j * EMB_DIM:(j + 1) * EMB_DIM], out)
    return out


# speedup vs baseline: 1.7528x; 1.7528x over previous
import functools

import jax
import jax.numpy as jnp
from jax import lax
from jax.experimental import pallas as pl
from jax.experimental.pallas import tpu as pltpu
from jax.experimental.pallas import tpu_sc as plsc

NUM_EMB = 100000
EMB_DIM = 32
BATCH = 16384

_G = 25088
_NROW = _G
_LSTEP = 1792
_TGRID = _G // _LSTEP

_INFO = plsc.get_sparse_core_info()
_NC = _INFO.num_cores
_NS = _INFO.num_subcores
_NW = _NC * _NS
_B_PER_W = BATCH // _NW
_GCHUNK = 128
_NGATHER = _B_PER_W // _GCHUNK


def _retile_body(t0, t1, t2, t3, out_ref):
    parts = [r[...].T for r in (t0, t1, t2, t3)]
    out_ref[...] = jnp.concatenate(parts, axis=1)


_retile = pl.pallas_call(
    _retile_body,
    grid=(_TGRID,),
    in_specs=[
        pl.BlockSpec((EMB_DIM, _LSTEP), functools.partial(lambda j, a: (0, j * _TGRID + a), j))
        for j in range(4)
    ],
    out_specs=pl.BlockSpec((_LSTEP, 128), lambda a: (a, 0)),
    out_shape=jax.ShapeDtypeStruct((_NROW, 128), jnp.float32),
)


@functools.partial(
    pl.kernel,
    mesh=plsc.VectorSubcoreMesh(core_axis_name="c", subcore_axis_name="s"),
    out_type=jax.ShapeDtypeStruct((BATCH, 128), jnp.float32),
    scratch_types=[
        pltpu.VMEM((_B_PER_W,), jnp.int32),
        pltpu.VMEM((_NGATHER, _GCHUNK), jnp.int32),
        pltpu.VMEM((_B_PER_W, 128), jnp.float32),
        pltpu.SemaphoreType.DMA,
    ],
)
def _embed_lookup(idx_hbm, tab_hbm, out_hbm, idx_v, q_v, rows_v, sem):
    wid = lax.axis_index("s") * _NC + lax.axis_index("c")
    base = wid * _B_PER_W
    pltpu.sync_copy(idx_hbm.at[pl.ds(base, _B_PER_W)], idx_v)

    for k in range(_B_PER_W // 16):
        v = idx_v[pl.ds(16 * k, 16)]
        q = jnp.where(
            v >= 3 * _G,
            v - 3 * _G,
            jnp.where(v >= 2 * _G, v - 2 * _G, jnp.where(v >= _G, v - _G, v)),
        )
        q_v[k // 8, pl.ds(16 * (k % 8), 16)] = q

    copies = [
        pltpu.async_copy(
            tab_hbm.at[q_v.at[j]], rows_v.at[pl.ds(_GCHUNK * j, _GCHUNK)], sem
        )
        for j in range(_NGATHER)
    ]
    for cp in copies:
        cp.wait()

    pltpu.sync_copy(rows_v, out_hbm.at[pl.ds(base, _B_PER_W)])


def kernel(char_indices, table):
    idx = char_indices.astype(jnp.int32)
    tt = table.T
    tab = _retile(tt, tt, tt, tt)
    wide = _embed_lookup(idx, tab)
    grp = (
        (idx >= _G).astype(jnp.int32)
        + (idx >= 2 * _G).astype(jnp.int32)
        + (idx >= 3 * _G).astype(jnp.int32)
    )[:, None]
    out = wide[:, 0:EMB_DIM]
    for j in range(1, 4):
        out = jnp.where(grp == j, wide[:, j * EMB_DIM:(j + 1) * EMB_DIM], out)
    return out


# speedup vs baseline: 2.3089x; 1.3173x over previous
import functools

import jax
import jax.numpy as jnp
from jax import lax
from jax.experimental import pallas as pl
from jax.experimental.pallas import tpu as pltpu
from jax.experimental.pallas import tpu_sc as plsc

NUM_EMB = 100000
EMB_DIM = 32
BATCH = 16384

_G = 25088
_NROW = _G
_LSTEP = 3584
_TGRID = _G // _LSTEP

_INFO = plsc.get_sparse_core_info()
_NC = _INFO.num_cores
_NS = _INFO.num_subcores
_NW = _NC * _NS
_B_PER_W = BATCH // _NW
_GCHUNK = 128
_NGATHER = _B_PER_W // _GCHUNK
_OSTRIDE = _B_PER_W + 1


def _retile_body(t0, t1, t2, t3, out_ref):
    parts = [r[...].T for r in (t0, t1, t2, t3)]
    out_ref[...] = jnp.concatenate(parts, axis=1)


_retile = pl.pallas_call(
    _retile_body,
    grid=(_TGRID,),
    in_specs=[
        pl.BlockSpec(
            (EMB_DIM, _LSTEP),
            functools.partial(lambda j, a: (0, j * _TGRID + a), j),
        )
        for j in range(4)
    ],
    out_specs=pl.BlockSpec((_LSTEP, 128), lambda a: (a, 0)),
    out_shape=jax.ShapeDtypeStruct((_NROW, 128), jnp.float32),
)


@functools.partial(
    pl.kernel,
    mesh=plsc.VectorSubcoreMesh(core_axis_name="c", subcore_axis_name="s"),
    out_type=jax.ShapeDtypeStruct((EMB_DIM, BATCH), jnp.float32),
    scratch_types=[
        pltpu.VMEM((_B_PER_W,), jnp.int32),
        pltpu.VMEM((_NGATHER, _GCHUNK), jnp.int32),
        pltpu.VMEM((_B_PER_W, 128), jnp.float32),
        pltpu.VMEM((EMB_DIM, _OSTRIDE), jnp.float32),
        pltpu.SemaphoreType.DMA,
    ],
    compiler_params=pltpu.CompilerParams(needs_layout_passes=False),
)
def _embed_lookup(idx_hbm, tab_hbm, out_hbm, idx_v, q_v, rows_v, outt_v, sem):
    wid = lax.axis_index("s") * _NC + lax.axis_index("c")
    base = wid * _B_PER_W
    pltpu.sync_copy(idx_hbm.at[pl.ds(base, _B_PER_W)], idx_v)

    for k in range(_B_PER_W // 16):
        v = idx_v[pl.ds(16 * k, 16)]
        q = jnp.where(
            v >= 3 * _G,
            v - 3 * _G,
            jnp.where(v >= 2 * _G, v - 2 * _G, jnp.where(v >= _G, v - _G, v)),
        )
        q_v[k // 8, pl.ds(16 * (k % 8), 16)] = q

    copies = [
        pltpu.async_copy(
            tab_hbm.at[q_v.at[j]], rows_v.at[pl.ds(_GCHUNK * j, _GCHUNK)], sem
        )
        for j in range(_NGATHER)
    ]
    for cp in copies:
        cp.wait()

    d_lo = lax.iota(jnp.int32, 16)
    d_hi = d_lo + 16

    def _chunk(k, carry):
        vidx = idx_v[pl.ds(16 * k, 16)]
        offv = jnp.where(
            vidx >= 3 * _G,
            96,
            jnp.where(vidx >= 2 * _G, 64, jnp.where(vidx >= _G, 32, 0)),
        )
        for t in range(16):
            b = 16 * k + t
            off = offv[t]
            x0 = rows_v[b, pl.ds(off, 16)]
            x1 = rows_v[b, pl.ds(off + 16, 16)]
            bvec = jnp.full((16,), b, jnp.int32)
            plsc.store_scatter(outt_v, [d_lo, bvec], x0)
            plsc.store_scatter(outt_v, [d_hi, bvec], x1)
        return carry

    lax.fori_loop(0, _B_PER_W // 16, _chunk, 0)

    pltpu.sync_copy(
        outt_v.at[:, pl.ds(0, _B_PER_W)], out_hbm.at[:, pl.ds(base, _B_PER_W)]
    )


def kernel(char_indices, table):
    idx = char_indices.astype(jnp.int32)
    tt = table.T
    tab = _retile(tt, tt, tt, tt)
    out_t = _embed_lookup(idx, tab)
    return out_t.T
